# baseline (speedup 1.0000x reference)
import functools
import math

import jax
import jax.numpy as jnp
from jax import lax
from jax.experimental import pallas as pl
from jax.experimental.pallas import tpu as pltpu

N = 10000
E = 160000
DIM = 256
HEADS = 16
HD = DIM // HEADS
B = 16
T = 32
RB = 1000
NBLK = N // RB


def _ln(x, w, b, eps=1e-5):
    m = jnp.mean(x, axis=-1, keepdims=True)
    va = jnp.mean((x - m) ** 2, axis=-1, keepdims=True)
    return (x - m) * jax.lax.rsqrt(va + eps) * w + b


def _gelu(x):
    return 0.5 * x * (1.0 + lax.erf(x * (1.0 / math.sqrt(2.0))))


def _ln_body(x_ref, w_ref, b_ref, o_ref):
    o_ref[...] = _ln(x_ref[...], w_ref[...], b_ref[...])


def _ln_x(x, w, b):
    return pl.pallas_call(
        _ln_body,
        grid=(NBLK,),
        in_specs=[
            pl.BlockSpec((RB, DIM), lambda i: (i, 0)),
            pl.BlockSpec((1, DIM), lambda i: (0, 0)),
            pl.BlockSpec((1, DIM), lambda i: (0, 0)),
        ],
        out_specs=pl.BlockSpec((RB, DIM), lambda i: (i, 0)),
        out_shape=jax.ShapeDtypeStruct((N, DIM), jnp.float32),
    )(x, w.reshape(1, DIM), b.reshape(1, DIM))


def _mlp_body(eps_ref, x_ref, xn_ref, aggr_ref, batch_ref,
              w1_ref, b1_ref, lnw_ref, lnb_ref, w2_ref, b2_ref, g_ref,
              x1_ref, sums_ref, cnts_ref):
    i = pl.program_id(0)
    eps = eps_ref[0]
    h = (1.0 + eps) * xn_ref[...] + aggr_ref[...]
    h = jnp.dot(h, w1_ref[...], preferred_element_type=jnp.float32) + b1_ref[...]
    h = _ln(h, lnw_ref[...], lnb_ref[...])
    h = _gelu(h)
    h = jnp.dot(h, w2_ref[...], preferred_element_type=jnp.float32) + b2_ref[...]
    x1 = x_ref[...] + g_ref[...] * h
    x1_ref[...] = x1

    bl = batch_ref[0]
    ohT = (lax.broadcasted_iota(jnp.int32, (B, RB), 0) == bl).astype(jnp.float32)

    @pl.when(i == 0)
    def _():
        sums_ref[...] = jnp.zeros_like(sums_ref)
        cnts_ref[...] = jnp.zeros_like(cnts_ref)

    sums_ref[...] += jnp.dot(ohT, x1, preferred_element_type=jnp.float32)
    cnts_ref[...] += jnp.dot(ohT, jnp.ones((RB, DIM), jnp.float32),
                             preferred_element_type=jnp.float32)


def _mlp_stage(eps, x, xn, aggr, batch3, w1t, b1, lnw, lnb, w2t, b2, g):
    return pl.pallas_call(
        _mlp_body,
        grid=(NBLK,),
        in_specs=[
            pl.BlockSpec(memory_space=pltpu.SMEM),
            pl.BlockSpec((RB, DIM), lambda i: (i, 0)),
            pl.BlockSpec((RB, DIM), lambda i: (i, 0)),
            pl.BlockSpec((RB, DIM), lambda i: (i, 0)),
            pl.BlockSpec((1, 1, RB), lambda i: (i, 0, 0)),
            pl.BlockSpec((DIM, 2 * DIM), lambda i: (0, 0)),
            pl.BlockSpec((1, 2 * DIM), lambda i: (0, 0)),
            pl.BlockSpec((1, 2 * DIM), lambda i: (0, 0)),
            pl.BlockSpec((1, 2 * DIM), lambda i: (0, 0)),
            pl.BlockSpec((2 * DIM, DIM), lambda i: (0, 0)),
            pl.BlockSpec((1, DIM), lambda i: (0, 0)),
            pl.BlockSpec((1, DIM), lambda i: (0, 0)),
        ],
        out_specs=[
            pl.BlockSpec((RB, DIM), lambda i: (i, 0)),
            pl.BlockSpec((B, DIM), lambda i: (0, 0)),
            pl.BlockSpec((B, DIM), lambda i: (0, 0)),
        ],
        out_shape=[
            jax.ShapeDtypeStruct((N, DIM), jnp.float32),
            jax.ShapeDtypeStruct((B, DIM), jnp.float32),
            jax.ShapeDtypeStruct((B, DIM), jnp.float32),
        ],
    )(eps, x, xn, aggr, batch3, w1t, b1, lnw, lnb, w2t, b2, g)


def _tok_body(vf_ref, sums_ref, cnts_ref,
              vnw_ref, vnb_ref, wq_ref, bq_ref, wk_ref, bk_ref, wv_ref, bv_ref,
              wo_ref, bo_ref, g_ref, kvw_ref, kvb_ref,
              vout_ref, kbd_ref, vbd_ref):
    ns = sums_ref[...] / jnp.maximum(cnts_ref[...], 1.0)
    vf = vf_ref[...]
    vn = _ln(vf, vnw_ref[...], vnb_ref[...])
    q = jnp.dot(vn, wq_ref[...], preferred_element_type=jnp.float32) + bq_ref[...]
    k_self = jnp.dot(vn, wk_ref[...], preferred_element_type=jnp.float32) + bk_ref[...]
    v_self = jnp.dot(vn, wv_ref[...], preferred_element_type=jnp.float32) + bv_ref[...]
    k_ns = jnp.dot(ns, wk_ref[...], preferred_element_type=jnp.float32) + bk_ref[...]
    v_ns = jnp.dot(ns, wv_ref[...], preferred_element_type=jnp.float32) + bv_ref[...]

    G = B * HEADS
    qh = q.reshape(B, T, HEADS, HD).transpose(0, 2, 1, 3).reshape(G, T, HD)
    kh_s = k_self.reshape(B, T, HEADS, HD).transpose(0, 2, 1, 3).reshape(G, T, HD)
    vh_s = v_self.reshape(B, T, HEADS, HD).transpose(0, 2, 1, 3).reshape(G, T, HD)
    kh_n = k_ns.reshape(B, 1, HEADS, HD).transpose(0, 2, 1, 3).reshape(G, 1, HD)
    vh_n = v_ns.reshape(B, 1, HEADS, HD).transpose(0, 2, 1, 3).reshape(G, 1, HD)
    kh = jnp.concatenate([kh_s, kh_n], axis=1)
    vh = jnp.concatenate([vh_s, vh_n], axis=1)
    att = jnp.einsum('gqd,gkd->gqk', qh, kh,
                     preferred_element_type=jnp.float32) * (1.0 / math.sqrt(HD))
    att = jax.nn.softmax(att, axis=-1)
    o = jnp.einsum('gqk,gkd->gqd', att, vh,
                   preferred_element_type=jnp.float32)
    o = o.reshape(B, HEADS, T, HD).transpose(0, 2, 1, 3).reshape(B * T, DIM)
    v_upd = jnp.dot(o, wo_ref[...], preferred_element_type=jnp.float32) + bo_ref[...]
    v_new = vf + g_ref[...] * v_upd
    vout_ref[...] = v_new

    kv = jnp.dot(v_new, kvw_ref[...], preferred_element_type=jnp.float32) + kvb_ref[...]
    k2 = kv[:, :DIM].reshape(B, T, DIM)
    v2 = kv[:, DIM:].reshape(B, T, DIM)
    hmask = (lax.broadcasted_iota(jnp.int32, (DIM, HEADS * T), 0) // HD ==
             lax.broadcasted_iota(jnp.int32, (DIM, HEADS * T), 1) // T
             ).astype(jnp.float32)
    k2t = k2.transpose(0, 2, 1)
    kbd_ref[...] = jnp.tile(k2t, (1, 1, HEADS)) * hmask
    hmask2 = (lax.broadcasted_iota(jnp.int32, (HEADS * T, DIM), 1) // HD ==
              lax.broadcasted_iota(jnp.int32, (HEADS * T, DIM), 0) // T
              ).astype(jnp.float32)
    vbd_ref[...] = jnp.tile(v2, (1, HEADS, 1)) * hmask2


def _tok_stage(vf, sums, cnts, vnw, vnb, wqt, bq, wkt, bk, wvt, bv,
               wot, bo, g, kvwt, kvb):
    full = lambda s: pl.BlockSpec(s, lambda: tuple(0 for _ in s))
    args = (vf, sums, cnts, vnw, vnb, wqt, bq, wkt, bk, wvt, bv, wot, bo, g,
            kvwt, kvb)
    return pl.pallas_call(
        _tok_body,
        in_specs=[full(a.shape) for a in args],
        out_specs=[full((B * T, DIM)), full((B, DIM, HEADS * T)),
                   full((B, HEADS * T, DIM))],
        out_shape=[
            jax.ShapeDtypeStruct((B * T, DIM), jnp.float32),
            jax.ShapeDtypeStruct((B, DIM, HEADS * T), jnp.float32),
            jax.ShapeDtypeStruct((B, HEADS * T, DIM), jnp.float32),
        ],
    )(*args)


def _xattn_body(present_ref, x1_ref, oh_ref, kbd_ref, vbd_ref,
                n3w_ref, n3b_ref, qw_ref, qb_ref, ow_ref, ob_ref, g_ref,
                n4w_ref, n4b_ref, f1w_ref, f1b_ref, bnw_ref, bnb_ref,
                f2w_ref, f2b_ref, xo_ref, sacc_ref, oacc_ref):
    i = pl.program_id(0)
    x1 = x1_ref[...]
    xn = _ln(x1, n3w_ref[...], n3b_ref[...])
    q2 = jnp.dot(xn, qw_ref[...], preferred_element_type=jnp.float32) + qb_ref[...]

    scale = 1.0 / math.sqrt(HD)
    sacc_ref[...] = jnp.zeros((RB, HEADS * T), jnp.float32)
    for b in range(B):
        @pl.when(present_ref[i, b] > 0)
        def _(b=b):
            rm = oh_ref[:, b:b + 1]
            sb = jnp.dot(q2, kbd_ref[b], preferred_element_type=jnp.float32)
            sacc_ref[...] += sb * rm
    scores = sacc_ref[...] * scale
    rmax = jnp.max(scores, axis=1, keepdims=True)
    e = jnp.exp(scores - rmax)
    gsel = (lax.broadcasted_iota(jnp.int32, (HEADS * T, HEADS), 0) // T ==
            lax.broadcasted_iota(jnp.int32, (HEADS * T, HEADS), 1)
            ).astype(jnp.float32)
    gselT = (lax.broadcasted_iota(jnp.int32, (HEADS, HEADS * T), 1) // T ==
             lax.broadcasted_iota(jnp.int32, (HEADS, HEADS * T), 0)
             ).astype(jnp.float32)
    denom = jnp.dot(e, gsel, preferred_element_type=jnp.float32)
    dfull = jnp.dot(denom, gselT, preferred_element_type=jnp.float32)
    attn = e / dfull

    oacc_ref[...] = jnp.zeros((RB, DIM), jnp.float32)
    for b in range(B):
        @pl.when(present_ref[i, b] > 0)
        def _(b=b):
            rm = oh_ref[:, b:b + 1]
            oacc_ref[...] += jnp.dot(attn * rm, vbd_ref[b],
                                     preferred_element_type=jnp.float32)
    out = oacc_ref[...]
    attn_out = jnp.dot(out, ow_ref[...], preferred_element_type=jnp.float32) + ob_ref[...]
    x2 = x1 + g_ref[...] * attn_out

    x4 = _ln(x2, n4w_ref[...], n4b_ref[...])
    f = jnp.dot(x4, f1w_ref[...], preferred_element_type=jnp.float32) + f1b_ref[...]
    f = f * (1.0 / math.sqrt(1.0 + 1e-5)) * bnw_ref[...] + bnb_ref[...]
    f = _gelu(f)
    f = jnp.dot(f, f2w_ref[...], preferred_element_type=jnp.float32) + f2b_ref[...]
    xo_ref[...] = x2 + f


def _xattn_stage(present, x1, ohN, kbd, vbd, n3w, n3b, qwt, qb, owt, obb, g,
                 n4w, n4b, f1wt, f1b, bnw, bnb, f2wt, f2b):
    rep = lambda s: pl.BlockSpec(s, lambda i: tuple(0 for _ in s))
    return pl.pallas_call(
        _xattn_body,
        grid=(NBLK,),
        in_specs=[
            pl.BlockSpec(memory_space=pltpu.SMEM),
            pl.BlockSpec((RB, DIM), lambda i: (i, 0)),
            pl.BlockSpec((RB, B), lambda i: (i, 0)),
            rep((B, DIM, HEADS * T)),
            rep((B, HEADS * T, DIM)),
            rep((1, DIM)), rep((1, DIM)),
            rep((DIM, DIM)), rep((1, DIM)),
            rep((DIM, DIM)), rep((1, DIM)), rep((1, DIM)),
            rep((1, DIM)), rep((1, DIM)),
            rep((DIM, 4 * DIM)), rep((1, 4 * DIM)),
            rep((1, 4 * DIM)), rep((1, 4 * DIM)),
            rep((4 * DIM, DIM)), rep((1, DIM)),
        ],
        out_specs=pl.BlockSpec((RB, DIM), lambda i: (i, 0)),
        out_shape=jax.ShapeDtypeStruct((N, DIM), jnp.float32),
        scratch_shapes=[pltpu.VMEM((RB, HEADS * T), jnp.float32),
                        pltpu.VMEM((RB, DIM), jnp.float32)],
    )(present, x1, ohN, kbd, vbd, n3w, n3b, qwt, qb, owt, obb, g,
      n4w, n4b, f1wt, f1b, bnw, bnb, f2wt, f2b)


from jax.experimental.pallas import tpu_sc as plsc

NC = 2
NS = 16
L = 16
HALF = N // NC
TRASH = HALF
CE = 320
NCHUNK = E // CE


def _gine_sc_body(xnr_hbm, src_hbm, dst_hbm, ea_hbm, out_hbm,
                  acc, rows_v, ea_v, srcv, dstv, gixv, sem):
    c = lax.axis_index("c")
    s = lax.axis_index("s")
    zero = jnp.zeros((L,), jnp.float32)

    def zfill(r, carry):
        acc[pl.ds(r * L, L)] = zero
        return carry

    lax.fori_loop(0, HALF + 8, zfill, 0, unroll=8)

    half0 = c * HALF
    col0 = s * L

    def chunk(k, carry):
        base = k * CE
        pltpu.sync_copy(src_hbm.at[pl.ds(base, CE)], srcv)
        pltpu.sync_copy(dst_hbm.at[pl.ds(base, CE)], dstv)
        for j in range(CE // L):
            gixv[pl.ds(j * L, L)] = srcv[pl.ds(j * L, L)] * L + s
        gather = pltpu.async_copy(xnr_hbm.at[gixv], rows_v, sem)
        pltpu.sync_copy(ea_hbm.at[pl.ds(base, CE), pl.ds(col0, L)], ea_v)
        gather.wait()

        cols = lax.iota(jnp.int32, L)

        def blk(b, cr):
            dv = dstv[pl.ds(b * L, L)]
            dl = dv - half0
            dl = jnp.where((dl >= 0) & (dl < HALF), dl, TRASH)
            dl = dl * L
            for j in range(L):
                e = b * L + j
                msg = jnp.maximum(rows_v[e] + ea_v[e], 0.0)
                addr = dl.at[jnp.full((L,), j, jnp.int32)].get(
                    mode='promise_in_bounds') + cols
                plsc.addupdate_scatter(acc, [addr], msg)
            return cr

        lax.fori_loop(0, CE // L, blk, 0)
        return carry

    lax.fori_loop(0, NCHUNK, chunk, 0)
    pltpu.sync_copy(acc.at[pl.ds(0, HALF * L)], out_hbm.at[c, s])


def _gine_aggr(xn, src, dst, edge_attr):
    xnr = xn.reshape(N * L, L)
    mesh = plsc.VectorSubcoreMesh(core_axis_name="c", subcore_axis_name="s",
                                  num_cores=NC, num_subcores=NS)
    f = pl.kernel(
        _gine_sc_body,
        out_type=jax.ShapeDtypeStruct((NC, NS, HALF * L), jnp.float32),
        mesh=mesh,
        scratch_types=[
            pltpu.VMEM(((HALF + 8) * L,), jnp.float32),
            pltpu.VMEM((CE, L), jnp.float32),
            pltpu.VMEM((CE, L), jnp.float32),
            pltpu.VMEM((CE,), jnp.int32),
            pltpu.VMEM((CE,), jnp.int32),
            pltpu.VMEM((CE,), jnp.int32),
            pltpu.SemaphoreType.DMA,
        ],
        compiler_params=pltpu.CompilerParams(use_tc_tiling_on_sc=False,
                                             needs_layout_passes=False),
    )
    out3 = f(xnr, src, dst, edge_attr)
    return (out3.reshape(NC, NS, HALF, L).transpose(0, 2, 1, 3)
            .reshape(N, DIM))


def kernel(x, edge_index, edge_attr, v, batch, params):
    p = params
    xn = _ln_x(x, p['norm1_w'], p['norm1_b'])
    aggr = _gine_aggr(xn, edge_index[0], edge_index[1], edge_attr)

    batch3 = batch.reshape(NBLK, 1, RB)
    r = lambda a: a.reshape(1, -1)
    x1, sums, cnts = _mlp_stage(
        p['eps'].reshape(1), x, xn, aggr, batch3,
        p['mlp1_w'].T, r(p['mlp1_b']), r(p['mlp_ln_w']), r(p['mlp_ln_b']),
        p['mlp2_w'].T, r(p['mlp2_b']), r(p['res1_g']))

    aw = p['attn_in_w']
    ab = p['attn_in_b']
    vf = v.reshape(B * T, DIM)
    v_out, kbd, vbd = _tok_stage(
        vf, sums, cnts, r(p['vnorm_w']), r(p['vnorm_b']),
        aw[:DIM].T, r(ab[:DIM]), aw[DIM:2 * DIM].T, r(ab[DIM:2 * DIM]),
        aw[2 * DIM:].T, r(ab[2 * DIM:]),
        p['attn_out_w'].T, r(p['attn_out_b']), r(p['res2_g']),
        p['kv_w'].T, r(p['kv_b']))

    ohN = (batch[:, None] == jnp.arange(B)[None, :]).astype(jnp.float32)
    blk = jnp.arange(N) // RB
    present = jnp.zeros((NBLK, B), jnp.int32).at[blk, batch].add(1)

    x_out = _xattn_stage(
        present, x1, ohN, kbd, vbd,
        r(p['norm3_w']), r(p['norm3_b']), p['q_w'].T, r(p['q_b']),
        p['o_w'].T, r(p['o_b']), r(p['res3_g']),
        r(p['norm4_w']), r(p['norm4_b']), p['ffn1_w'].T, r(p['ffn1_b']),
        r(p['bn_w']), r(p['bn_b']), p['ffn2_w'].T, r(p['ffn2_b']))
    return x_out, v_out.reshape(B, T, DIM)


# speedup vs baseline: 1.1866x; 1.1866x over previous
import functools
import math

import jax
import jax.numpy as jnp
from jax import lax
from jax.experimental import pallas as pl
from jax.experimental.pallas import tpu as pltpu

N = 10000
E = 160000
DIM = 256
HEADS = 16
HD = DIM // HEADS
B = 16
T = 32
RB = 1000
NBLK = N // RB


def _ln(x, w, b, eps=1e-5):
    m = jnp.mean(x, axis=-1, keepdims=True)
    va = jnp.mean((x - m) ** 2, axis=-1, keepdims=True)
    return (x - m) * jax.lax.rsqrt(va + eps) * w + b


def _gelu(x):
    return 0.5 * x * (1.0 + lax.erf(x * (1.0 / math.sqrt(2.0))))


def _ln_body(x_ref, w_ref, b_ref, o_ref):
    o_ref[...] = _ln(x_ref[...], w_ref[...], b_ref[...])


def _ln_x(x, w, b):
    return pl.pallas_call(
        _ln_body,
        grid=(NBLK,),
        in_specs=[
            pl.BlockSpec((RB, DIM), lambda i: (i, 0)),
            pl.BlockSpec((1, DIM), lambda i: (0, 0)),
            pl.BlockSpec((1, DIM), lambda i: (0, 0)),
        ],
        out_specs=pl.BlockSpec((RB, DIM), lambda i: (i, 0)),
        out_shape=jax.ShapeDtypeStruct((N, DIM), jnp.float32),
    )(x, w.reshape(1, DIM), b.reshape(1, DIM))


def _mlp_body(eps_ref, x_ref, xn_ref, aggr_ref, batch_ref,
              w1_ref, b1_ref, lnw_ref, lnb_ref, w2_ref, b2_ref, g_ref,
              x1_ref, sums_ref, cnts_ref):
    i = pl.program_id(0)
    eps = eps_ref[0]
    h = (1.0 + eps) * xn_ref[...] + aggr_ref[...]
    h = jnp.dot(h, w1_ref[...], preferred_element_type=jnp.float32) + b1_ref[...]
    h = _ln(h, lnw_ref[...], lnb_ref[...])
    h = _gelu(h)
    h = jnp.dot(h, w2_ref[...], preferred_element_type=jnp.float32) + b2_ref[...]
    x1 = x_ref[...] + g_ref[...] * h
    x1_ref[...] = x1

    bl = batch_ref[0]
    ohT = (lax.broadcasted_iota(jnp.int32, (B, RB), 0) == bl).astype(jnp.float32)

    @pl.when(i == 0)
    def _():
        sums_ref[...] = jnp.zeros_like(sums_ref)
        cnts_ref[...] = jnp.zeros_like(cnts_ref)

    sums_ref[...] += jnp.dot(ohT, x1, preferred_element_type=jnp.float32)
    cnts_ref[...] += jnp.dot(ohT, jnp.ones((RB, DIM), jnp.float32),
                             preferred_element_type=jnp.float32)


def _mlp_stage(eps, x, xn, aggr, batch3, w1t, b1, lnw, lnb, w2t, b2, g):
    return pl.pallas_call(
        _mlp_body,
        grid=(NBLK,),
        in_specs=[
            pl.BlockSpec(memory_space=pltpu.SMEM),
            pl.BlockSpec((RB, DIM), lambda i: (i, 0)),
            pl.BlockSpec((RB, DIM), lambda i: (i, 0)),
            pl.BlockSpec((RB, DIM), lambda i: (i, 0)),
            pl.BlockSpec((1, 1, RB), lambda i: (i, 0, 0)),
            pl.BlockSpec((DIM, 2 * DIM), lambda i: (0, 0)),
            pl.BlockSpec((1, 2 * DIM), lambda i: (0, 0)),
            pl.BlockSpec((1, 2 * DIM), lambda i: (0, 0)),
            pl.BlockSpec((1, 2 * DIM), lambda i: (0, 0)),
            pl.BlockSpec((2 * DIM, DIM), lambda i: (0, 0)),
            pl.BlockSpec((1, DIM), lambda i: (0, 0)),
            pl.BlockSpec((1, DIM), lambda i: (0, 0)),
        ],
        out_specs=[
            pl.BlockSpec((RB, DIM), lambda i: (i, 0)),
            pl.BlockSpec((B, DIM), lambda i: (0, 0)),
            pl.BlockSpec((B, DIM), lambda i: (0, 0)),
        ],
        out_shape=[
            jax.ShapeDtypeStruct((N, DIM), jnp.float32),
            jax.ShapeDtypeStruct((B, DIM), jnp.float32),
            jax.ShapeDtypeStruct((B, DIM), jnp.float32),
        ],
    )(eps, x, xn, aggr, batch3, w1t, b1, lnw, lnb, w2t, b2, g)


def _tok_body(vf_ref, sums_ref, cnts_ref,
              vnw_ref, vnb_ref, wq_ref, bq_ref, wk_ref, bk_ref, wv_ref, bv_ref,
              wo_ref, bo_ref, g_ref, kvw_ref, kvb_ref,
              vout_ref, kbd_ref, vbd_ref):
    ns = sums_ref[...] / jnp.maximum(cnts_ref[...], 1.0)
    vf = vf_ref[...]
    vn = _ln(vf, vnw_ref[...], vnb_ref[...])
    q = jnp.dot(vn, wq_ref[...], preferred_element_type=jnp.float32) + bq_ref[...]
    k_self = jnp.dot(vn, wk_ref[...], preferred_element_type=jnp.float32) + bk_ref[...]
    v_self = jnp.dot(vn, wv_ref[...], preferred_element_type=jnp.float32) + bv_ref[...]
    k_ns = jnp.dot(ns, wk_ref[...], preferred_element_type=jnp.float32) + bk_ref[...]
    v_ns = jnp.dot(ns, wv_ref[...], preferred_element_type=jnp.float32) + bv_ref[...]

    G = B * HEADS
    qh = q.reshape(B, T, HEADS, HD).transpose(0, 2, 1, 3).reshape(G, T, HD)
    kh_s = k_self.reshape(B, T, HEADS, HD).transpose(0, 2, 1, 3).reshape(G, T, HD)
    vh_s = v_self.reshape(B, T, HEADS, HD).transpose(0, 2, 1, 3).reshape(G, T, HD)
    kh_n = k_ns.reshape(B, 1, HEADS, HD).transpose(0, 2, 1, 3).reshape(G, 1, HD)
    vh_n = v_ns.reshape(B, 1, HEADS, HD).transpose(0, 2, 1, 3).reshape(G, 1, HD)
    kh = jnp.concatenate([kh_s, kh_n], axis=1)
    vh = jnp.concatenate([vh_s, vh_n], axis=1)
    att = jnp.einsum('gqd,gkd->gqk', qh, kh,
                     preferred_element_type=jnp.float32) * (1.0 / math.sqrt(HD))
    att = jax.nn.softmax(att, axis=-1)
    o = jnp.einsum('gqk,gkd->gqd', att, vh,
                   preferred_element_type=jnp.float32)
    o = o.reshape(B, HEADS, T, HD).transpose(0, 2, 1, 3).reshape(B * T, DIM)
    v_upd = jnp.dot(o, wo_ref[...], preferred_element_type=jnp.float32) + bo_ref[...]
    v_new = vf + g_ref[...] * v_upd
    vout_ref[...] = v_new

    kv = jnp.dot(v_new, kvw_ref[...], preferred_element_type=jnp.float32) + kvb_ref[...]
    k2 = kv[:, :DIM].reshape(B, T, DIM)
    v2 = kv[:, DIM:].reshape(B, T, DIM)
    hmask = (lax.broadcasted_iota(jnp.int32, (DIM, HEADS * T), 0) // HD ==
             lax.broadcasted_iota(jnp.int32, (DIM, HEADS * T), 1) // T
             ).astype(jnp.float32)
    k2t = k2.transpose(0, 2, 1)
    kbd_ref[...] = jnp.tile(k2t, (1, 1, HEADS)) * hmask
    hmask2 = (lax.broadcasted_iota(jnp.int32, (HEADS * T, DIM), 1) // HD ==
              lax.broadcasted_iota(jnp.int32, (HEADS * T, DIM), 0) // T
              ).astype(jnp.float32)
    vbd_ref[...] = jnp.tile(v2, (1, HEADS, 1)) * hmask2


def _tok_stage(vf, sums, cnts, vnw, vnb, wqt, bq, wkt, bk, wvt, bv,
               wot, bo, g, kvwt, kvb):
    full = lambda s: pl.BlockSpec(s, lambda: tuple(0 for _ in s))
    args = (vf, sums, cnts, vnw, vnb, wqt, bq, wkt, bk, wvt, bv, wot, bo, g,
            kvwt, kvb)
    return pl.pallas_call(
        _tok_body,
        in_specs=[full(a.shape) for a in args],
        out_specs=[full((B * T, DIM)), full((B, DIM, HEADS * T)),
                   full((B, HEADS * T, DIM))],
        out_shape=[
            jax.ShapeDtypeStruct((B * T, DIM), jnp.float32),
            jax.ShapeDtypeStruct((B, DIM, HEADS * T), jnp.float32),
            jax.ShapeDtypeStruct((B, HEADS * T, DIM), jnp.float32),
        ],
    )(*args)


def _xattn_body(present_ref, x1_ref, oh_ref, kbd_ref, vbd_ref,
                n3w_ref, n3b_ref, qw_ref, qb_ref, ow_ref, ob_ref, g_ref,
                n4w_ref, n4b_ref, f1w_ref, f1b_ref, bnw_ref, bnb_ref,
                f2w_ref, f2b_ref, xo_ref, sacc_ref, oacc_ref):
    i = pl.program_id(0)
    x1 = x1_ref[...]
    xn = _ln(x1, n3w_ref[...], n3b_ref[...])
    q2 = jnp.dot(xn, qw_ref[...], preferred_element_type=jnp.float32) + qb_ref[...]

    scale = 1.0 / math.sqrt(HD)
    sacc_ref[...] = jnp.zeros((RB, HEADS * T), jnp.float32)
    for b in range(B):
        @pl.when(present_ref[i, b] > 0)
        def _(b=b):
            rm = oh_ref[:, b:b + 1]
            sb = jnp.dot(q2, kbd_ref[b], preferred_element_type=jnp.float32)
            sacc_ref[...] += sb * rm
    scores = sacc_ref[...] * scale
    rmax = jnp.max(scores, axis=1, keepdims=True)
    e = jnp.exp(scores - rmax)
    gsel = (lax.broadcasted_iota(jnp.int32, (HEADS * T, HEADS), 0) // T ==
            lax.broadcasted_iota(jnp.int32, (HEADS * T, HEADS), 1)
            ).astype(jnp.float32)
    gselT = (lax.broadcasted_iota(jnp.int32, (HEADS, HEADS * T), 1) // T ==
             lax.broadcasted_iota(jnp.int32, (HEADS, HEADS * T), 0)
             ).astype(jnp.float32)
    denom = jnp.dot(e, gsel, preferred_element_type=jnp.float32)
    dfull = jnp.dot(denom, gselT, preferred_element_type=jnp.float32)
    attn = e / dfull

    oacc_ref[...] = jnp.zeros((RB, DIM), jnp.float32)
    for b in range(B):
        @pl.when(present_ref[i, b] > 0)
        def _(b=b):
            rm = oh_ref[:, b:b + 1]
            oacc_ref[...] += jnp.dot(attn * rm, vbd_ref[b],
                                     preferred_element_type=jnp.float32)
    out = oacc_ref[...]
    attn_out = jnp.dot(out, ow_ref[...], preferred_element_type=jnp.float32) + ob_ref[...]
    x2 = x1 + g_ref[...] * attn_out

    x4 = _ln(x2, n4w_ref[...], n4b_ref[...])
    f = jnp.dot(x4, f1w_ref[...], preferred_element_type=jnp.float32) + f1b_ref[...]
    f = f * (1.0 / math.sqrt(1.0 + 1e-5)) * bnw_ref[...] + bnb_ref[...]
    f = _gelu(f)
    f = jnp.dot(f, f2w_ref[...], preferred_element_type=jnp.float32) + f2b_ref[...]
    xo_ref[...] = x2 + f


def _xattn_stage(present, x1, ohN, kbd, vbd, n3w, n3b, qwt, qb, owt, obb, g,
                 n4w, n4b, f1wt, f1b, bnw, bnb, f2wt, f2b):
    rep = lambda s: pl.BlockSpec(s, lambda i: tuple(0 for _ in s))
    return pl.pallas_call(
        _xattn_body,
        grid=(NBLK,),
        in_specs=[
            pl.BlockSpec(memory_space=pltpu.SMEM),
            pl.BlockSpec((RB, DIM), lambda i: (i, 0)),
            pl.BlockSpec((RB, B), lambda i: (i, 0)),
            rep((B, DIM, HEADS * T)),
            rep((B, HEADS * T, DIM)),
            rep((1, DIM)), rep((1, DIM)),
            rep((DIM, DIM)), rep((1, DIM)),
            rep((DIM, DIM)), rep((1, DIM)), rep((1, DIM)),
            rep((1, DIM)), rep((1, DIM)),
            rep((DIM, 4 * DIM)), rep((1, 4 * DIM)),
            rep((1, 4 * DIM)), rep((1, 4 * DIM)),
            rep((4 * DIM, DIM)), rep((1, DIM)),
        ],
        out_specs=pl.BlockSpec((RB, DIM), lambda i: (i, 0)),
        out_shape=jax.ShapeDtypeStruct((N, DIM), jnp.float32),
        scratch_shapes=[pltpu.VMEM((RB, HEADS * T), jnp.float32),
                        pltpu.VMEM((RB, DIM), jnp.float32)],
    )(present, x1, ohN, kbd, vbd, n3w, n3b, qwt, qb, owt, obb, g,
      n4w, n4b, f1wt, f1b, bnw, bnb, f2wt, f2b)


from jax.experimental.pallas import tpu_sc as plsc

NC = 2
NS = 16
L = 16
HALF = N // NC
TRASH = HALF
CE = 640
NCHUNK = E // CE


def _gine_sc_body(xnr_hbm, src_hbm, dst_hbm, ea_hbm, out_hbm,
                  acc, rows_v, ea_v, srcv, dstv, gixv, sem):
    c = lax.axis_index("c")
    s = lax.axis_index("s")
    zero = jnp.zeros((L,), jnp.float32)

    def zfill(r, carry):
        acc[pl.ds(r * L, L)] = zero
        return carry

    lax.fori_loop(0, HALF + 8, zfill, 0, unroll=8)

    half0 = c * HALF
    col0 = s * L

    def chunk(k, carry):
        base = k * CE
        pltpu.sync_copy(src_hbm.at[pl.ds(base, CE)], srcv)
        pltpu.sync_copy(dst_hbm.at[pl.ds(base, CE)], dstv)
        for j in range(CE // L):
            gixv[pl.ds(j * L, L)] = srcv[pl.ds(j * L, L)] * L + s
        gather = pltpu.async_copy(xnr_hbm.at[gixv], rows_v, sem)
        pltpu.sync_copy(ea_hbm.at[pl.ds(base, CE), pl.ds(col0, L)], ea_v)
        gather.wait()

        cols = lax.iota(jnp.int32, L)

        def blk(b, cr):
            dv = dstv[pl.ds(b * L, L)]
            dl = dv - half0
            dl = jnp.where((dl >= 0) & (dl < HALF), dl, TRASH)
            dl = dl * L
            for j in range(L):
                e = b * L + j
                msg = jnp.maximum(rows_v[e] + ea_v[e], 0.0)
                addr = dl.at[jnp.full((L,), j, jnp.int32)].get(
                    mode='promise_in_bounds') + cols
                plsc.addupdate_scatter(acc, [addr], msg)
            return cr

        lax.fori_loop(0, CE // L, blk, 0, unroll=2)
        return carry

    lax.fori_loop(0, NCHUNK, chunk, 0)
    pltpu.sync_copy(acc.at[pl.ds(0, HALF * L)], out_hbm.at[c, s])


def _gine_aggr(xn, src, dst, edge_attr):
    xnr = xn.reshape(N * L, L)
    mesh = plsc.VectorSubcoreMesh(core_axis_name="c", subcore_axis_name="s",
                                  num_cores=NC, num_subcores=NS)
    f = pl.kernel(
        _gine_sc_body,
        out_type=jax.ShapeDtypeStruct((NC, NS, HALF * L), jnp.float32),
        mesh=mesh,
        scratch_types=[
            pltpu.VMEM(((HALF + 8) * L,), jnp.float32),
            pltpu.VMEM((CE, L), jnp.float32),
            pltpu.VMEM((CE, L), jnp.float32),
            pltpu.VMEM((CE,), jnp.int32),
            pltpu.VMEM((CE,), jnp.int32),
            pltpu.VMEM((CE,), jnp.int32),
            pltpu.SemaphoreType.DMA,
        ],
        compiler_params=pltpu.CompilerParams(use_tc_tiling_on_sc=False,
                                             needs_layout_passes=False),
    )
    out3 = f(xnr, src, dst, edge_attr)
    return (out3.reshape(NC, NS, HALF, L).transpose(0, 2, 1, 3)
            .reshape(N, DIM))


def kernel(x, edge_index, edge_attr, v, batch, params):
    p = params
    xn = _ln_x(x, p['norm1_w'], p['norm1_b'])
    aggr = _gine_aggr(xn, edge_index[0], edge_index[1], edge_attr)

    batch3 = batch.reshape(NBLK, 1, RB)
    r = lambda a: a.reshape(1, -1)
    x1, sums, cnts = _mlp_stage(
        p['eps'].reshape(1), x, xn, aggr, batch3,
        p['mlp1_w'].T, r(p['mlp1_b']), r(p['mlp_ln_w']), r(p['mlp_ln_b']),
        p['mlp2_w'].T, r(p['mlp2_b']), r(p['res1_g']))

    aw = p['attn_in_w']
    ab = p['attn_in_b']
    vf = v.reshape(B * T, DIM)
    v_out, kbd, vbd = _tok_stage(
        vf, sums, cnts, r(p['vnorm_w']), r(p['vnorm_b']),
        aw[:DIM].T, r(ab[:DIM]), aw[DIM:2 * DIM].T, r(ab[DIM:2 * DIM]),
        aw[2 * DIM:].T, r(ab[2 * DIM:]),
        p['attn_out_w'].T, r(p['attn_out_b']), r(p['res2_g']),
        p['kv_w'].T, r(p['kv_b']))

    ohN = (batch[:, None] == jnp.arange(B)[None, :]).astype(jnp.float32)
    blk = jnp.arange(N) // RB
    present = jnp.zeros((NBLK, B), jnp.int32).at[blk, batch].add(1)

    x_out = _xattn_stage(
        present, x1, ohN, kbd, vbd,
        r(p['norm3_w']), r(p['norm3_b']), p['q_w'].T, r(p['q_b']),
        p['o_w'].T, r(p['o_b']), r(p['res3_g']),
        r(p['norm4_w']), r(p['norm4_b']), p['ffn1_w'].T, r(p['ffn1_b']),
        r(p['bn_w']), r(p['bn_b']), p['ffn2_w'].T, r(p['ffn2_b']))
    return x_out, v_out.reshape(B, T, DIM)


# speedup vs baseline: 1.5505x; 1.3066x over previous
import functools
import math

import jax
import jax.numpy as jnp
from jax import lax
from jax.experimental import pallas as pl
from jax.experimental.pallas import tpu as pltpu

N = 10000
E = 160000
DIM = 256
HEADS = 16
HD = DIM // HEADS
B = 16
T = 32
RB = 1000
NBLK = N // RB


def _ln(x, w, b, eps=1e-5):
    m = jnp.mean(x, axis=-1, keepdims=True)
    va = jnp.mean((x - m) ** 2, axis=-1, keepdims=True)
    return (x - m) * jax.lax.rsqrt(va + eps) * w + b


def _gelu(x):
    return 0.5 * x * (1.0 + lax.erf(x * (1.0 / math.sqrt(2.0))))


def _ln_body(x_ref, w_ref, b_ref, o_ref):
    o_ref[...] = _ln(x_ref[...], w_ref[...], b_ref[...])


def _ln_x(x, w, b):
    return pl.pallas_call(
        _ln_body,
        grid=(NBLK,),
        in_specs=[
            pl.BlockSpec((RB, DIM), lambda i: (i, 0)),
            pl.BlockSpec((1, DIM), lambda i: (0, 0)),
            pl.BlockSpec((1, DIM), lambda i: (0, 0)),
        ],
        out_specs=pl.BlockSpec((RB, DIM), lambda i: (i, 0)),
        out_shape=jax.ShapeDtypeStruct((N, DIM), jnp.float32),
    )(x, w.reshape(1, DIM), b.reshape(1, DIM))


def _mlp_body(eps_ref, x_ref, xn_ref, aggr_ref, batch_ref,
              w1_ref, b1_ref, lnw_ref, lnb_ref, w2_ref, b2_ref, g_ref,
              x1_ref, sums_ref, cnts_ref):
    i = pl.program_id(0)
    eps = eps_ref[0]
    h = (1.0 + eps) * xn_ref[...] + aggr_ref[...]
    h = jnp.dot(h, w1_ref[...], preferred_element_type=jnp.float32) + b1_ref[...]
    h = _ln(h, lnw_ref[...], lnb_ref[...])
    h = _gelu(h)
    h = jnp.dot(h, w2_ref[...], preferred_element_type=jnp.float32) + b2_ref[...]
    x1 = x_ref[...] + g_ref[...] * h
    x1_ref[...] = x1

    bl = batch_ref[0]
    ohT = (lax.broadcasted_iota(jnp.int32, (B, RB), 0) == bl).astype(jnp.float32)

    @pl.when(i == 0)
    def _():
        sums_ref[...] = jnp.zeros_like(sums_ref)
        cnts_ref[...] = jnp.zeros_like(cnts_ref)

    sums_ref[...] += jnp.dot(ohT, x1, preferred_element_type=jnp.float32)
    cnts_ref[...] += jnp.dot(ohT, jnp.ones((RB, DIM), jnp.float32),
                             preferred_element_type=jnp.float32)


def _mlp_stage(eps, x, xn, aggr, batch3, w1t, b1, lnw, lnb, w2t, b2, g):
    return pl.pallas_call(
        _mlp_body,
        grid=(NBLK,),
        in_specs=[
            pl.BlockSpec(memory_space=pltpu.SMEM),
            pl.BlockSpec((RB, DIM), lambda i: (i, 0)),
            pl.BlockSpec((RB, DIM), lambda i: (i, 0)),
            pl.BlockSpec((RB, DIM), lambda i: (i, 0)),
            pl.BlockSpec((1, 1, RB), lambda i: (i, 0, 0)),
            pl.BlockSpec((DIM, 2 * DIM), lambda i: (0, 0)),
            pl.BlockSpec((1, 2 * DIM), lambda i: (0, 0)),
            pl.BlockSpec((1, 2 * DIM), lambda i: (0, 0)),
            pl.BlockSpec((1, 2 * DIM), lambda i: (0, 0)),
            pl.BlockSpec((2 * DIM, DIM), lambda i: (0, 0)),
            pl.BlockSpec((1, DIM), lambda i: (0, 0)),
            pl.BlockSpec((1, DIM), lambda i: (0, 0)),
        ],
        out_specs=[
            pl.BlockSpec((RB, DIM), lambda i: (i, 0)),
            pl.BlockSpec((B, DIM), lambda i: (0, 0)),
            pl.BlockSpec((B, DIM), lambda i: (0, 0)),
        ],
        out_shape=[
            jax.ShapeDtypeStruct((N, DIM), jnp.float32),
            jax.ShapeDtypeStruct((B, DIM), jnp.float32),
            jax.ShapeDtypeStruct((B, DIM), jnp.float32),
        ],
    )(eps, x, xn, aggr, batch3, w1t, b1, lnw, lnb, w2t, b2, g)


def _tok_body(vf_ref, sums_ref, cnts_ref,
              vnw_ref, vnb_ref, wq_ref, bq_ref, wk_ref, bk_ref, wv_ref, bv_ref,
              wo_ref, bo_ref, g_ref, kvw_ref, kvb_ref,
              vout_ref, kbd_ref, vbd_ref):
    ns = sums_ref[...] / jnp.maximum(cnts_ref[...], 1.0)
    vf = vf_ref[...]
    vn = _ln(vf, vnw_ref[...], vnb_ref[...])
    q = jnp.dot(vn, wq_ref[...], preferred_element_type=jnp.float32) + bq_ref[...]
    k_self = jnp.dot(vn, wk_ref[...], preferred_element_type=jnp.float32) + bk_ref[...]
    v_self = jnp.dot(vn, wv_ref[...], preferred_element_type=jnp.float32) + bv_ref[...]
    k_ns = jnp.dot(ns, wk_ref[...], preferred_element_type=jnp.float32) + bk_ref[...]
    v_ns = jnp.dot(ns, wv_ref[...], preferred_element_type=jnp.float32) + bv_ref[...]

    G = B * HEADS
    qh = q.reshape(B, T, HEADS, HD).transpose(0, 2, 1, 3).reshape(G, T, HD)
    kh_s = k_self.reshape(B, T, HEADS, HD).transpose(0, 2, 1, 3).reshape(G, T, HD)
    vh_s = v_self.reshape(B, T, HEADS, HD).transpose(0, 2, 1, 3).reshape(G, T, HD)
    kh_n = k_ns.reshape(B, 1, HEADS, HD).transpose(0, 2, 1, 3).reshape(G, 1, HD)
    vh_n = v_ns.reshape(B, 1, HEADS, HD).transpose(0, 2, 1, 3).reshape(G, 1, HD)
    kh = jnp.concatenate([kh_s, kh_n], axis=1)
    vh = jnp.concatenate([vh_s, vh_n], axis=1)
    att = jnp.einsum('gqd,gkd->gqk', qh, kh,
                     preferred_element_type=jnp.float32) * (1.0 / math.sqrt(HD))
    att = jax.nn.softmax(att, axis=-1)
    o = jnp.einsum('gqk,gkd->gqd', att, vh,
                   preferred_element_type=jnp.float32)
    o = o.reshape(B, HEADS, T, HD).transpose(0, 2, 1, 3).reshape(B * T, DIM)
    v_upd = jnp.dot(o, wo_ref[...], preferred_element_type=jnp.float32) + bo_ref[...]
    v_new = vf + g_ref[...] * v_upd
    vout_ref[...] = v_new

    kv = jnp.dot(v_new, kvw_ref[...], preferred_element_type=jnp.float32) + kvb_ref[...]
    k2 = kv[:, :DIM].reshape(B, T, DIM)
    v2 = kv[:, DIM:].reshape(B, T, DIM)
    hmask = (lax.broadcasted_iota(jnp.int32, (DIM, HEADS * T), 0) // HD ==
             lax.broadcasted_iota(jnp.int32, (DIM, HEADS * T), 1) // T
             ).astype(jnp.float32)
    k2t = k2.transpose(0, 2, 1)
    kbd_ref[...] = jnp.tile(k2t, (1, 1, HEADS)) * hmask
    hmask2 = (lax.broadcasted_iota(jnp.int32, (HEADS * T, DIM), 1) // HD ==
              lax.broadcasted_iota(jnp.int32, (HEADS * T, DIM), 0) // T
              ).astype(jnp.float32)
    vbd_ref[...] = jnp.tile(v2, (1, HEADS, 1)) * hmask2


def _tok_stage(vf, sums, cnts, vnw, vnb, wqt, bq, wkt, bk, wvt, bv,
               wot, bo, g, kvwt, kvb):
    full = lambda s: pl.BlockSpec(s, lambda: tuple(0 for _ in s))
    args = (vf, sums, cnts, vnw, vnb, wqt, bq, wkt, bk, wvt, bv, wot, bo, g,
            kvwt, kvb)
    return pl.pallas_call(
        _tok_body,
        in_specs=[full(a.shape) for a in args],
        out_specs=[full((B * T, DIM)), full((B, DIM, HEADS * T)),
                   full((B, HEADS * T, DIM))],
        out_shape=[
            jax.ShapeDtypeStruct((B * T, DIM), jnp.float32),
            jax.ShapeDtypeStruct((B, DIM, HEADS * T), jnp.float32),
            jax.ShapeDtypeStruct((B, HEADS * T, DIM), jnp.float32),
        ],
    )(*args)


def _xattn_body(present_ref, x1_ref, oh_ref, kbd_ref, vbd_ref,
                n3w_ref, n3b_ref, qw_ref, qb_ref, ow_ref, ob_ref, g_ref,
                n4w_ref, n4b_ref, f1w_ref, f1b_ref, bnw_ref, bnb_ref,
                f2w_ref, f2b_ref, xo_ref, sacc_ref, oacc_ref):
    i = pl.program_id(0)
    x1 = x1_ref[...]
    xn = _ln(x1, n3w_ref[...], n3b_ref[...])
    q2 = jnp.dot(xn, qw_ref[...], preferred_element_type=jnp.float32) + qb_ref[...]

    scale = 1.0 / math.sqrt(HD)
    sacc_ref[...] = jnp.zeros((RB, HEADS * T), jnp.float32)
    for b in range(B):
        @pl.when(present_ref[i, b] > 0)
        def _(b=b):
            rm = oh_ref[:, b:b + 1]
            sb = jnp.dot(q2, kbd_ref[b], preferred_element_type=jnp.float32)
            sacc_ref[...] += sb * rm
    scores = sacc_ref[...] * scale
    rmax = jnp.max(scores, axis=1, keepdims=True)
    e = jnp.exp(scores - rmax)
    gsel = (lax.broadcasted_iota(jnp.int32, (HEADS * T, HEADS), 0) // T ==
            lax.broadcasted_iota(jnp.int32, (HEADS * T, HEADS), 1)
            ).astype(jnp.float32)
    gselT = (lax.broadcasted_iota(jnp.int32, (HEADS, HEADS * T), 1) // T ==
             lax.broadcasted_iota(jnp.int32, (HEADS, HEADS * T), 0)
             ).astype(jnp.float32)
    denom = jnp.dot(e, gsel, preferred_element_type=jnp.float32)
    dfull = jnp.dot(denom, gselT, preferred_element_type=jnp.float32)
    attn = e / dfull

    oacc_ref[...] = jnp.zeros((RB, DIM), jnp.float32)
    for b in range(B):
        @pl.when(present_ref[i, b] > 0)
        def _(b=b):
            rm = oh_ref[:, b:b + 1]
            oacc_ref[...] += jnp.dot(attn * rm, vbd_ref[b],
                                     preferred_element_type=jnp.float32)
    out = oacc_ref[...]
    attn_out = jnp.dot(out, ow_ref[...], preferred_element_type=jnp.float32) + ob_ref[...]
    x2 = x1 + g_ref[...] * attn_out

    x4 = _ln(x2, n4w_ref[...], n4b_ref[...])
    f = jnp.dot(x4, f1w_ref[...], preferred_element_type=jnp.float32) + f1b_ref[...]
    f = f * (1.0 / math.sqrt(1.0 + 1e-5)) * bnw_ref[...] + bnb_ref[...]
    f = _gelu(f)
    f = jnp.dot(f, f2w_ref[...], preferred_element_type=jnp.float32) + f2b_ref[...]
    xo_ref[...] = x2 + f


def _xattn_stage(present, x1, ohN, kbd, vbd, n3w, n3b, qwt, qb, owt, obb, g,
                 n4w, n4b, f1wt, f1b, bnw, bnb, f2wt, f2b):
    rep = lambda s: pl.BlockSpec(s, lambda i: tuple(0 for _ in s))
    return pl.pallas_call(
        _xattn_body,
        grid=(NBLK,),
        in_specs=[
            pl.BlockSpec(memory_space=pltpu.SMEM),
            pl.BlockSpec((RB, DIM), lambda i: (i, 0)),
            pl.BlockSpec((RB, B), lambda i: (i, 0)),
            rep((B, DIM, HEADS * T)),
            rep((B, HEADS * T, DIM)),
            rep((1, DIM)), rep((1, DIM)),
            rep((DIM, DIM)), rep((1, DIM)),
            rep((DIM, DIM)), rep((1, DIM)), rep((1, DIM)),
            rep((1, DIM)), rep((1, DIM)),
            rep((DIM, 4 * DIM)), rep((1, 4 * DIM)),
            rep((1, 4 * DIM)), rep((1, 4 * DIM)),
            rep((4 * DIM, DIM)), rep((1, DIM)),
        ],
        out_specs=pl.BlockSpec((RB, DIM), lambda i: (i, 0)),
        out_shape=jax.ShapeDtypeStruct((N, DIM), jnp.float32),
        scratch_shapes=[pltpu.VMEM((RB, HEADS * T), jnp.float32),
                        pltpu.VMEM((RB, DIM), jnp.float32)],
    )(present, x1, ohN, kbd, vbd, n3w, n3b, qwt, qb, owt, obb, g,
      n4w, n4b, f1wt, f1b, bnw, bnb, f2wt, f2b)


from jax.experimental.pallas import tpu_sc as plsc

NC = 2
NS = 16
L = 16
HALF = N // NC
TRASH = HALF
CE = 640
NCHUNK = E // CE


def _gine_sc_body(xnr_hbm, src_hbm, dst_hbm, ea_hbm, out_hbm,
                  acc, rows0, rows1, ea0, ea1, src0, src1, dst0, dst1,
                  gix0, gix1, semi0, semi1, semd0, semd1):
    c = lax.axis_index("c")
    s = lax.axis_index("s")
    zero = jnp.zeros((L,), jnp.float32)

    def zfill(r, carry):
        acc[pl.ds(r * L, L)] = zero
        return carry

    lax.fori_loop(0, HALF + 8, zfill, 0, unroll=8)

    half0 = c * HALF
    col0 = s * L
    rowsb = [rows0, rows1]
    eab = [ea0, ea1]
    srcb = [src0, src1]
    dstb = [dst0, dst1]
    gixb = [gix0, gix1]
    semi = [semi0, semi1]
    semd = [semd0, semd1]

    def fire_idx(k, t):
        base = k * CE
        pltpu.async_copy(src_hbm.at[pl.ds(base, CE)], srcb[t], semi[t])
        pltpu.async_copy(dst_hbm.at[pl.ds(base, CE)], dstb[t], semi[t])

    def wait_idx(k, t):
        base = k * CE
        pltpu.make_async_copy(src_hbm.at[pl.ds(base, CE)], srcb[t], semi[t]).wait()
        pltpu.make_async_copy(dst_hbm.at[pl.ds(base, CE)], dstb[t], semi[t]).wait()

    def fire_dat(k, t):
        for j in range(CE // L):
            gixb[t][pl.ds(j * L, L)] = srcb[t][pl.ds(j * L, L)] * L + s
        pltpu.async_copy(xnr_hbm.at[gixb[t]], rowsb[t], semd[t])
        base = k * CE
        pltpu.async_copy(ea_hbm.at[pl.ds(base, CE), pl.ds(col0, L)], eab[t], semd[t])

    def wait_dat(k, t):
        base = k * CE
        pltpu.make_async_copy(xnr_hbm.at[gixb[t]], rowsb[t], semd[t]).wait()
        pltpu.make_async_copy(ea_hbm.at[pl.ds(base, CE), pl.ds(col0, L)],
                              eab[t], semd[t]).wait()

    cols = lax.iota(jnp.int32, L)

    def compute(t):
        rv, ev, dv_ref = rowsb[t], eab[t], dstb[t]

        def blk(bq, cr):
            dv = dv_ref[pl.ds(bq * L, L)]
            dl = dv - half0
            dl = jnp.where((dl >= 0) & (dl < HALF), dl, TRASH) * L
            for j in range(L):
                e = bq * L + j
                msg = jnp.maximum(rv[e] + ev[e], 0.0)
                addr = dl.at[jnp.full((L,), j, jnp.int32)].get(
                    mode='promise_in_bounds') + cols
                plsc.addupdate_scatter(acc, [addr], msg)
            return cr

        lax.fori_loop(0, CE // L, blk, 0, unroll=2)

    fire_idx(0, 0)
    wait_idx(0, 0)
    fire_dat(0, 0)
    fire_idx(1, 1)

    def pair(m, carry):
        k0 = 2 * m
        k1 = k0 + 1
        wait_idx(k1, 1)
        fire_dat(k1, 1)
        wait_dat(k0, 0)
        compute(0)

        @pl.when(k0 + 2 < NCHUNK)
        def _():
            fire_idx(k0 + 2, 0)

        @pl.when(k0 + 2 < NCHUNK)
        def _():
            wait_idx(k0 + 2, 0)
            fire_dat(k0 + 2, 0)

        wait_dat(k1, 1)
        compute(1)

        @pl.when(k1 + 2 < NCHUNK)
        def _():
            fire_idx(k1 + 2, 1)

        return carry

    lax.fori_loop(0, NCHUNK // 2, pair, 0)
    pltpu.sync_copy(acc.at[pl.ds(0, HALF * L)], out_hbm.at[c, s])


def _gine_aggr(xn, src, dst, edge_attr):
    xnr = xn.reshape(N * L, L)
    mesh = plsc.VectorSubcoreMesh(core_axis_name="c", subcore_axis_name="s",
                                  num_cores=NC, num_subcores=NS)
    f = pl.kernel(
        _gine_sc_body,
        out_type=jax.ShapeDtypeStruct((NC, NS, HALF * L), jnp.float32),
        mesh=mesh,
        scratch_types=[
            pltpu.VMEM(((HALF + 8) * L,), jnp.float32),
            pltpu.VMEM((CE, L), jnp.float32),
            pltpu.VMEM((CE, L), jnp.float32),
            pltpu.VMEM((CE, L), jnp.float32),
            pltpu.VMEM((CE, L), jnp.float32),
            pltpu.VMEM((CE,), jnp.int32),
            pltpu.VMEM((CE,), jnp.int32),
            pltpu.VMEM((CE,), jnp.int32),
            pltpu.VMEM((CE,), jnp.int32),
            pltpu.VMEM((CE,), jnp.int32),
            pltpu.VMEM((CE,), jnp.int32),
            pltpu.SemaphoreType.DMA,
            pltpu.SemaphoreType.DMA,
            pltpu.SemaphoreType.DMA,
            pltpu.SemaphoreType.DMA,
        ],
        compiler_params=pltpu.CompilerParams(use_tc_tiling_on_sc=False,
                                             needs_layout_passes=False),
    )
    out3 = f(xnr, src, dst, edge_attr)
    return (out3.reshape(NC, NS, HALF, L).transpose(0, 2, 1, 3)
            .reshape(N, DIM))


def kernel(x, edge_index, edge_attr, v, batch, params):
    p = params
    xn = _ln_x(x, p['norm1_w'], p['norm1_b'])
    aggr = _gine_aggr(xn, edge_index[0], edge_index[1], edge_attr)

    batch3 = batch.reshape(NBLK, 1, RB)
    r = lambda a: a.reshape(1, -1)
    x1, sums, cnts = _mlp_stage(
        p['eps'].reshape(1), x, xn, aggr, batch3,
        p['mlp1_w'].T, r(p['mlp1_b']), r(p['mlp_ln_w']), r(p['mlp_ln_b']),
        p['mlp2_w'].T, r(p['mlp2_b']), r(p['res1_g']))

    aw = p['attn_in_w']
    ab = p['attn_in_b']
    vf = v.reshape(B * T, DIM)
    v_out, kbd, vbd = _tok_stage(
        vf, sums, cnts, r(p['vnorm_w']), r(p['vnorm_b']),
        aw[:DIM].T, r(ab[:DIM]), aw[DIM:2 * DIM].T, r(ab[DIM:2 * DIM]),
        aw[2 * DIM:].T, r(ab[2 * DIM:]),
        p['attn_out_w'].T, r(p['attn_out_b']), r(p['res2_g']),
        p['kv_w'].T, r(p['kv_b']))

    ohN = (batch[:, None] == jnp.arange(B)[None, :]).astype(jnp.float32)
    blk = jnp.arange(N) // RB
    present = jnp.zeros((NBLK, B), jnp.int32).at[blk, batch].add(1)

    x_out = _xattn_stage(
        present, x1, ohN, kbd, vbd,
        r(p['norm3_w']), r(p['norm3_b']), p['q_w'].T, r(p['q_b']),
        p['o_w'].T, r(p['o_b']), r(p['res3_g']),
        r(p['norm4_w']), r(p['norm4_b']), p['ffn1_w'].T, r(p['ffn1_b']),
        r(p['bn_w']), r(p['bn_b']), p['ffn2_w'].T, r(p['ffn2_b']))
    return x_out, v_out.reshape(B, T, DIM)
